# baseline (speedup 1.0000x reference)
import dataclasses
import functools

import jax
import jax.numpy as jnp
from jax import lax
from jax.experimental import pallas as pl
from jax.experimental.pallas import tpu as pltpu
from jax.experimental.pallas import tpu_sc as plsc

EMBED_DIM = 64
PAD_DIM = 128
NUM_CORES = 2
NUM_SUBCORES = 16
NUM_WORKERS = NUM_CORES * NUM_SUBCORES
CHUNK_B = 256


@functools.partial(jax.jit, static_argnames=("batch", "hist"))
def _sc_lookup(idx_t, table128, batch, hist):
    bpw = batch // NUM_WORKERS
    sub = bpw // CHUNK_B
    n_chunks = hist * sub
    mesh = plsc.VectorSubcoreMesh(core_axis_name="c", subcore_axis_name="s")
    cp = pltpu.CompilerParams()
    if "needs_layout_passes" in pltpu.CompilerParams.__dataclass_fields__:
        cp = dataclasses.replace(cp, needs_layout_passes=False)

    @functools.partial(
        pl.kernel,
        mesh=mesh,
        compiler_params=cp,
        out_type=jax.ShapeDtypeStruct((hist, EMBED_DIM, batch), jnp.float32),
        scratch_types=[
            pltpu.VMEM((CHUNK_B,), jnp.int32),
            pltpu.VMEM((CHUNK_B,), jnp.int32),
            pltpu.VMEM((CHUNK_B, PAD_DIM), jnp.float32),
            pltpu.VMEM((CHUNK_B, PAD_DIM), jnp.float32),
            pltpu.VMEM((EMBED_DIM, CHUNK_B), jnp.float32),
            pltpu.VMEM((EMBED_DIM, CHUNK_B), jnp.float32),
            pltpu.SemaphoreType.DMA,
            pltpu.SemaphoreType.DMA,
            pltpu.SemaphoreType.DMA,
            pltpu.SemaphoreType.DMA,
        ],
    )
    def k(
        idx_hbm, tab_hbm, out_hbm, ib0, ib1, rb0, rb1, ob0, ob1, s0, s1, w0, w1
    ):
        ibufs = (ib0, ib1)
        rbufs = (rb0, rb1)
        obufs = (ob0, ob1)
        gsems = (s0, s1)
        wsems = (w0, w1)
        wid = lax.axis_index("s") * NUM_CORES + lax.axis_index("c")
        b_base = wid * bpw
        iota = lax.iota(jnp.int32, 16)
        jvecs = [iota + j0 for j0 in range(0, CHUNK_B, 16)]

        def fire(t, slot):
            h = t // sub
            boff = (t % sub) * CHUNK_B
            pltpu.sync_copy(
                idx_hbm.at[h, pl.ds(b_base + boff, CHUNK_B)], ibufs[slot]
            )
            pltpu.async_copy(tab_hbm.at[ibufs[slot]], rbufs[slot], gsems[slot])

        fire(0, 0)

        @pl.loop(0, n_chunks // 2)
        def _(c):
            for b in range(2):
                t = c * 2 + b
                h = t // sub
                boff = (t % sub) * CHUNK_B

                @pl.when(t + 1 < n_chunks)
                def _():
                    fire(t + 1, (b + 1) % 2)

                rb = rbufs[b]
                ob = obufs[b]
                pltpu.make_async_copy(
                    tab_hbm.at[ibufs[b]], rb, gsems[b]
                ).wait()

                @pl.when(c >= 1)
                def _():
                    pltpu.make_async_copy(
                        ob,
                        out_hbm.at[0, :, pl.ds(b_base, CHUNK_B)],
                        wsems[b],
                    ).wait()

                @pl.loop(0, EMBED_DIM // 8)
                def _(dg):
                    d_base = dg * 8
                    for dd in range(8):
                        d = d_base + dd
                        d16 = jnp.full((16,), 0, jnp.int32) + d
                        vals = [plsc.load_gather(rb, [jv, d16]) for jv in jvecs]
                        for j, v in enumerate(vals):
                            ob[d, pl.ds(j * 16, 16)] = v

                pltpu.async_copy(
                    ob,
                    out_hbm.at[h, :, pl.ds(b_base + boff, CHUNK_B)],
                    wsems[b],
                )

        for b in range(2):
            pltpu.make_async_copy(
                obufs[b],
                out_hbm.at[0, :, pl.ds(b_base, CHUNK_B)],
                wsems[b],
            ).wait()

    return k(idx_t, table128)


def kernel(indices, in_embeddings):
    batch, hist = indices.shape
    table128 = jnp.pad(in_embeddings, ((0, 0), (0, PAD_DIM - EMBED_DIM)))
    out3 = _sc_lookup(indices.T, table128, batch, hist)
    return jnp.transpose(out3, (2, 0, 1))


# speedup vs baseline: 1.0409x; 1.0409x over previous
import dataclasses
import functools

import jax
import jax.numpy as jnp
from jax import lax
from jax.experimental import pallas as pl
from jax.experimental.pallas import tpu as pltpu
from jax.experimental.pallas import tpu_sc as plsc

EMBED_DIM = 64
PAD_DIM = 128
NUM_CORES = 2
NUM_SUBCORES = 16
NUM_WORKERS = NUM_CORES * NUM_SUBCORES
CHUNK_B = 256


@functools.partial(jax.jit, static_argnames=("batch", "hist"))
def _sc_lookup(idx_t, table128, batch, hist):
    bpw = batch // NUM_WORKERS
    sub = bpw // CHUNK_B
    n_chunks = hist * sub
    mesh = plsc.VectorSubcoreMesh(core_axis_name="c", subcore_axis_name="s")
    cp = pltpu.CompilerParams()
    if "needs_layout_passes" in pltpu.CompilerParams.__dataclass_fields__:
        cp = dataclasses.replace(cp, needs_layout_passes=False)

    @functools.partial(
        pl.kernel,
        mesh=mesh,
        compiler_params=cp,
        out_type=jax.ShapeDtypeStruct((hist, EMBED_DIM, batch), jnp.float32),
        scratch_types=[
            pltpu.VMEM((CHUNK_B,), jnp.int32),
            pltpu.VMEM((CHUNK_B,), jnp.int32),
            pltpu.VMEM((CHUNK_B, PAD_DIM), jnp.float32),
            pltpu.VMEM((CHUNK_B, PAD_DIM), jnp.float32),
            pltpu.VMEM((EMBED_DIM, CHUNK_B), jnp.float32),
            pltpu.VMEM((EMBED_DIM, CHUNK_B), jnp.float32),
            pltpu.SemaphoreType.DMA,
            pltpu.SemaphoreType.DMA,
            pltpu.SemaphoreType.DMA,
            pltpu.SemaphoreType.DMA,
        ],
    )
    def k(
        idx_hbm, tab_hbm, out_hbm, ib0, ib1, rb0, rb1, ob0, ob1, s0, s1, w0, w1
    ):
        ibufs = (ib0, ib1)
        rbufs = (rb0, rb1)
        obufs = (ob0, ob1)
        gsems = (s0, s1)
        wsems = (w0, w1)
        wid = lax.axis_index("s") * NUM_CORES + lax.axis_index("c")
        b_base = wid * bpw
        iota = lax.iota(jnp.int32, 16)
        jvecs = [iota + j0 for j0 in range(0, CHUNK_B, 16)]

        def fire(t, slot):
            h = t // sub
            boff = (t % sub) * CHUNK_B
            pltpu.sync_copy(
                idx_hbm.at[h, pl.ds(b_base + boff, CHUNK_B)], ibufs[slot]
            )
            pltpu.async_copy(tab_hbm.at[ibufs[slot]], rbufs[slot], gsems[slot])

        fire(0, 0)

        @pl.loop(0, n_chunks // 2)
        def _(c):
            for b in range(2):
                t = c * 2 + b
                h = t // sub
                boff = (t % sub) * CHUNK_B

                @pl.when(t + 1 < n_chunks)
                def _():
                    fire(t + 1, (b + 1) % 2)

                rb = rbufs[b]
                ob = obufs[b]
                pltpu.make_async_copy(
                    tab_hbm.at[ibufs[b]], rb, gsems[b]
                ).wait()

                @pl.when(c >= 1)
                def _():
                    pltpu.make_async_copy(
                        ob,
                        out_hbm.at[0, :, pl.ds(b_base, CHUNK_B)],
                        wsems[b],
                    ).wait()

                @plsc.parallel_loop(0, EMBED_DIM, unroll=8)
                def _(d):
                    d16 = jnp.full((16,), 0, jnp.int32) + d
                    vals = [plsc.load_gather(rb, [jv, d16]) for jv in jvecs]
                    for j, v in enumerate(vals):
                        ob[d, pl.ds(j * 16, 16)] = v

                pltpu.async_copy(
                    ob,
                    out_hbm.at[h, :, pl.ds(b_base + boff, CHUNK_B)],
                    wsems[b],
                )

        for b in range(2):
            pltpu.make_async_copy(
                obufs[b],
                out_hbm.at[0, :, pl.ds(b_base, CHUNK_B)],
                wsems[b],
            ).wait()

    return k(idx_t, table128)


def kernel(indices, in_embeddings):
    batch, hist = indices.shape
    table128 = jnp.pad(in_embeddings, ((0, 0), (0, PAD_DIM - EMBED_DIM)))
    out3 = _sc_lookup(indices.T, table128, batch, hist)
    return jnp.transpose(out3, (2, 0, 1))


# speedup vs baseline: 1.7389x; 1.6705x over previous
import dataclasses
import functools

import jax
import jax.numpy as jnp
from jax import lax
from jax.experimental import pallas as pl
from jax.experimental.pallas import tpu as pltpu
from jax.experimental.pallas import tpu_sc as plsc

EMBED_DIM = 64
PAD_DIM = 128
NUM_CORES = 2
NUM_SUBCORES = 16
NUM_WORKERS = NUM_CORES * NUM_SUBCORES
CHUNK_B = 256


@functools.partial(jax.jit, static_argnames=("batch", "hist"))
def _sc_lookup(idx_t, table128, batch, hist):
    bpw = batch // NUM_WORKERS
    sub = bpw // CHUNK_B
    n_chunks = hist * sub
    mesh = plsc.VectorSubcoreMesh(core_axis_name="c", subcore_axis_name="s")
    cp = pltpu.CompilerParams()
    if "needs_layout_passes" in pltpu.CompilerParams.__dataclass_fields__:
        cp = dataclasses.replace(cp, needs_layout_passes=False)

    @functools.partial(
        pl.kernel,
        mesh=mesh,
        compiler_params=cp,
        out_type=jax.ShapeDtypeStruct((hist, EMBED_DIM, batch), jnp.float32),
        scratch_types=[
            pltpu.VMEM((CHUNK_B,), jnp.int32),
            pltpu.VMEM((CHUNK_B,), jnp.int32),
            pltpu.VMEM((CHUNK_B, PAD_DIM), jnp.float32),
            pltpu.VMEM((CHUNK_B, PAD_DIM), jnp.float32),
            pltpu.VMEM((EMBED_DIM, CHUNK_B), jnp.float32),
            pltpu.VMEM((EMBED_DIM, CHUNK_B), jnp.float32),
            pltpu.SemaphoreType.DMA,
            pltpu.SemaphoreType.DMA,
            pltpu.SemaphoreType.DMA,
            pltpu.SemaphoreType.DMA,
        ],
    )
    def k(
        idx_hbm, tab_hbm, out_hbm, ib0, ib1, rb0, rb1, ob0, ob1, s0, s1, w0, w1
    ):
        ibufs = (ib0, ib1)
        rbufs = (rb0, rb1)
        obufs = (ob0, ob1)
        gsems = (s0, s1)
        wsems = (w0, w1)
        wid = lax.axis_index("s") * NUM_CORES + lax.axis_index("c")
        b_base = wid * bpw
        iota = lax.iota(jnp.int32, 16)
        jvecs = [iota + j0 for j0 in range(0, CHUNK_B, 16)]
        rots = [(iota + k) & 15 for k in range(16)]

        def fire(t, slot):
            h = t // sub
            boff = (t % sub) * CHUNK_B
            pltpu.sync_copy(
                idx_hbm.at[h, pl.ds(b_base + boff, CHUNK_B)], ibufs[slot]
            )
            pltpu.async_copy(tab_hbm.at[ibufs[slot]], rbufs[slot], gsems[slot])

        fire(0, 0)

        @pl.loop(0, n_chunks // 2)
        def _(c):
            for b in range(2):
                t = c * 2 + b
                h = t // sub
                boff = (t % sub) * CHUNK_B

                @pl.when(t + 1 < n_chunks)
                def _():
                    fire(t + 1, (b + 1) % 2)

                rb = rbufs[b]
                ob = obufs[b]
                pltpu.make_async_copy(
                    tab_hbm.at[ibufs[b]], rb, gsems[b]
                ).wait()

                @pl.when(c >= 1)
                def _():
                    pltpu.make_async_copy(
                        ob,
                        out_hbm.at[0, :, pl.ds(b_base, CHUNK_B)],
                        wsems[b],
                    ).wait()

                @plsc.parallel_loop(0, CHUNK_B // 16, unroll=2)
                def _(jb):
                    jidx = jvecs[0] + jb * 16
                    for d0 in range(0, EMBED_DIM, 16):
                        for k in range(16):
                            didx = rots[k] + d0
                            v = plsc.load_gather(rb, [jidx, didx])
                            plsc.store_scatter(ob, [didx, jidx], v)

                pltpu.async_copy(
                    ob,
                    out_hbm.at[h, :, pl.ds(b_base + boff, CHUNK_B)],
                    wsems[b],
                )

        for b in range(2):
            pltpu.make_async_copy(
                obufs[b],
                out_hbm.at[0, :, pl.ds(b_base, CHUNK_B)],
                wsems[b],
            ).wait()

    return k(idx_t, table128)


def kernel(indices, in_embeddings):
    batch, hist = indices.shape
    table128 = jnp.pad(in_embeddings, ((0, 0), (0, PAD_DIM - EMBED_DIM)))
    out3 = _sc_lookup(indices.T, table128, batch, hist)
    return jnp.transpose(out3, (2, 0, 1))
